# baseline (speedup 1.0000x reference)
import functools

import jax
import jax.numpy as jnp
from jax.experimental import pallas as pl

_TI = 32


def _row_onehot(TI, V):
    r = jax.lax.broadcasted_iota(jnp.int32, (TI * V, TI), 0) // V
    c = jax.lax.broadcasted_iota(jnp.int32, (TI * V, TI), 1)
    return (r == c).astype(jnp.float32)


def _ec_body(TI, xf_ref, xt_ref, adjp_ref, e_ref,
             Wab_ref, Wb_ref, Wc_ref, b1_ref, W2_ref, b2_ref, y_ref):
    V = xf_ref.shape[1]
    H = W2_ref.shape[1]
    C2 = e_ref.shape[3]
    A_t = xt_ref[0] @ Wab_ref[...] + b1_ref[...]
    Bn = xf_ref[0] @ Wb_ref[...]
    E = e_ref[0].reshape(TI * V, C2) @ Wc_ref[...]
    At = _row_onehot(TI, V) @ A_t
    Bt = jnp.broadcast_to(Bn[None], (TI, V, H)).reshape(TI * V, H)
    h1 = jnp.maximum(E + At + Bt, 0.0)
    h2 = jnp.maximum(h1 @ W2_ref[...] + b2_ref[...], 0.0)
    pen = jnp.where(adjp_ref[0] > 0.0, 0.0, jnp.float32(-1e30))
    hm = h2 + pen
    agg = jnp.max(hm.reshape(TI, V, H), axis=1)
    y_ref[0] = jnp.where(agg <= -1e29, 0.0, agg)


def _edge_conv(adjp, x, e, W1, b1, W2, b2):
    B, V, C = x.shape
    C2 = e.shape[-1]
    H = W2.shape[1]
    Wab = W1[:C] - W1[C:2 * C]
    W1b = W1[C:2 * C]
    W1c = W1[2 * C:]
    TI = _TI
    return pl.pallas_call(
        functools.partial(_ec_body, TI),
        grid=(B, V // TI),
        in_specs=[
            pl.BlockSpec((1, V, C), lambda b, i: (b, 0, 0)),
            pl.BlockSpec((1, TI, C), lambda b, i: (b, i, 0)),
            pl.BlockSpec((1, TI * V, 1), lambda b, i: (b, i, 0)),
            pl.BlockSpec((1, TI, V, C2), lambda b, i: (b, i, 0, 0)),
            pl.BlockSpec((C, H), lambda b, i: (0, 0)),
            pl.BlockSpec((C, H), lambda b, i: (0, 0)),
            pl.BlockSpec((C2, H), lambda b, i: (0, 0)),
            pl.BlockSpec((1, H), lambda b, i: (0, 0)),
            pl.BlockSpec((H, H), lambda b, i: (0, 0)),
            pl.BlockSpec((1, H), lambda b, i: (0, 0)),
        ],
        out_specs=pl.BlockSpec((1, TI, H), lambda b, i: (b, i, 0)),
        out_shape=jax.ShapeDtypeStruct((B, V, H), jnp.float32),
    )(x, x, adjp, e, Wab, W1b, W1c, b1.reshape(1, H), W2, b2.reshape(1, H))


def _fin_body(TI, zf_ref, zt_ref, Wa_ref, Wb_ref, b3_ref, W4_ref, b4_ref,
              out_ref):
    V = zf_ref.shape[1]
    H2 = W4_ref.shape[0]
    P = zf_ref[0] @ Wa_ref[...]
    Q = zt_ref[0] @ Wb_ref[...] + b3_ref[...]
    Pt = jnp.broadcast_to(P[None], (TI, V, H2)).reshape(TI * V, H2)
    Qt = _row_onehot(TI, V) @ Q
    h = jnp.maximum(Pt + Qt, 0.0)
    o = h @ W4_ref[...] + b4_ref[...]
    out_ref[0] = jax.nn.sigmoid(o).reshape(TI, V, 1)


def _edge_predict(z, lin3_W, lin3_b, out_W, out_b):
    B, V, C = z.shape
    H2 = lin3_W.shape[1]
    Wa = lin3_W[:C]
    Wb = lin3_W[C:]
    TI = _TI
    out4 = pl.pallas_call(
        functools.partial(_fin_body, TI),
        grid=(B, V // TI),
        in_specs=[
            pl.BlockSpec((1, V, C), lambda b, i: (b, 0, 0)),
            pl.BlockSpec((1, TI, C), lambda b, i: (b, i, 0)),
            pl.BlockSpec((C, H2), lambda b, i: (0, 0)),
            pl.BlockSpec((C, H2), lambda b, i: (0, 0)),
            pl.BlockSpec((1, H2), lambda b, i: (0, 0)),
            pl.BlockSpec((H2, 1), lambda b, i: (0, 0)),
            pl.BlockSpec((1, 1), lambda b, i: (0, 0)),
        ],
        out_specs=pl.BlockSpec((1, TI, V, 1), lambda b, i: (b, i, 0, 0)),
        out_shape=jax.ShapeDtypeStruct((B, V, V, 1), jnp.float32),
    )(z, z, Wa, Wb, lin3_b.reshape(1, H2), out_W, out_b.reshape(1, 1))
    return out4.reshape(B, V, V)


def kernel(adjacency, node_features, edge_attributes,
           ec1_W1, ec1_b1, ec1_W2, ec1_b2,
           ec2_W1, ec2_b1, ec2_W2, ec2_b2,
           lin3_W, lin3_b, out_W, out_b):
    B, V, _ = node_features.shape
    adjp = adjacency.reshape(B, V * V, 1)
    y = _edge_conv(adjp, node_features, edge_attributes,
                   ec1_W1, ec1_b1, ec1_W2, ec1_b2)
    z = _edge_conv(adjp, y, edge_attributes,
                   ec2_W1, ec2_b1, ec2_W2, ec2_b2)
    return _edge_predict(z, lin3_W, lin3_b, out_W, out_b)


# speedup vs baseline: 1.1107x; 1.1107x over previous
import functools

import jax
import jax.numpy as jnp
from jax.experimental import pallas as pl

_TI = 32


def _ec_body(TI, xf_ref, xt_ref, ea_ref, oh_ref,
             Wab_ref, Wb_ref, Wc5_ref, b1_ref, W2_ref, b2_ref, y_ref):
    V = xf_ref.shape[1]
    H = W2_ref.shape[1]
    C5 = ea_ref.shape[3]
    A_t = xt_ref[0] @ Wab_ref[...] + b1_ref[...]
    Bn = xf_ref[0] @ Wb_ref[...]
    ea = ea_ref[0]
    E = ea.reshape(TI * V, C5) @ Wc5_ref[...]
    At = oh_ref[...] @ A_t
    Bt = jnp.broadcast_to(Bn[None], (TI, V, H)).reshape(TI * V, H)
    h1 = jnp.maximum(E + At + Bt, 0.0)
    h2 = jnp.maximum(h1 @ W2_ref[...] + b2_ref[...], 0.0)
    pen = jnp.where(ea[:, :, C5 - 1:C5] > 0.0, 0.0, jnp.float32(-1e30))
    hm = h2.reshape(TI, V, H) + pen
    agg = jnp.max(hm, axis=1)
    y_ref[0] = jnp.where(agg <= -1e29, 0.0, agg)


def _edge_conv(ea, oh, x, W1, b1, W2, b2):
    B, V, C = x.shape
    C5 = ea.shape[-1]
    H = W2.shape[1]
    Wab = W1[:C] - W1[C:2 * C]
    W1b = W1[C:2 * C]
    Wc5 = jnp.concatenate([W1[2 * C:], jnp.zeros((1, H), jnp.float32)], axis=0)
    TI = _TI
    return pl.pallas_call(
        functools.partial(_ec_body, TI),
        grid=(B, V // TI),
        in_specs=[
            pl.BlockSpec((1, V, C), lambda b, i: (b, 0, 0)),
            pl.BlockSpec((1, TI, C), lambda b, i: (b, i, 0)),
            pl.BlockSpec((1, TI, V, C5), lambda b, i: (b, i, 0, 0)),
            pl.BlockSpec((TI * V, TI), lambda b, i: (0, 0)),
            pl.BlockSpec((C, H), lambda b, i: (0, 0)),
            pl.BlockSpec((C, H), lambda b, i: (0, 0)),
            pl.BlockSpec((C5, H), lambda b, i: (0, 0)),
            pl.BlockSpec((1, H), lambda b, i: (0, 0)),
            pl.BlockSpec((H, H), lambda b, i: (0, 0)),
            pl.BlockSpec((1, H), lambda b, i: (0, 0)),
        ],
        out_specs=pl.BlockSpec((1, TI, H), lambda b, i: (b, i, 0)),
        out_shape=jax.ShapeDtypeStruct((B, V, H), jnp.float32),
    )(x, x, ea, oh, Wab, W1b, Wc5, b1.reshape(1, H), W2, b2.reshape(1, H))


def _fin_body(TI, zT_ref, zt_ref, oht_ref, ohh_ref,
              WaT_ref, Wb_ref, b3_ref, W4sel_ref, b4_ref, out_ref):
    V = zT_ref.shape[2]
    H2 = Wb_ref.shape[1]
    PT = WaT_ref[...] @ zT_ref[0]
    Q = zt_ref[0] @ Wb_ref[...] + b3_ref[...]
    R1 = oht_ref[...] @ Q
    Qcol = (R1 * ohh_ref[...]) @ jnp.ones((H2, 1), jnp.float32)
    PTt = jnp.broadcast_to(PT[None], (TI, H2, V)).reshape(TI * H2, V)
    Hb = jnp.maximum(PTt + Qcol, 0.0)
    ot = W4sel_ref[...] @ Hb + b4_ref[...]
    out_ref[0] = jax.nn.sigmoid(ot)


def _edge_predict(z, lin3_W, lin3_b, out_W, out_b):
    B, V, C = z.shape
    H2 = lin3_W.shape[1]
    TI = _TI
    zT = z.transpose(0, 2, 1)
    WaT = lin3_W[:C].T
    Wb = lin3_W[C:]
    r = jnp.arange(TI * H2)
    oht = (r[:, None] // H2 == jnp.arange(TI)[None, :]).astype(jnp.float32)
    ohh = (r[:, None] % H2 == jnp.arange(H2)[None, :]).astype(jnp.float32)
    W4sel = jnp.kron(jnp.eye(TI, dtype=jnp.float32), out_W.reshape(1, H2))
    return pl.pallas_call(
        functools.partial(_fin_body, TI),
        grid=(B, V // TI),
        in_specs=[
            pl.BlockSpec((1, C, V), lambda b, i: (b, 0, 0)),
            pl.BlockSpec((1, TI, C), lambda b, i: (b, i, 0)),
            pl.BlockSpec((TI * H2, TI), lambda b, i: (0, 0)),
            pl.BlockSpec((TI * H2, H2), lambda b, i: (0, 0)),
            pl.BlockSpec((H2, C), lambda b, i: (0, 0)),
            pl.BlockSpec((C, H2), lambda b, i: (0, 0)),
            pl.BlockSpec((1, H2), lambda b, i: (0, 0)),
            pl.BlockSpec((TI, TI * H2), lambda b, i: (0, 0)),
            pl.BlockSpec((1, 1), lambda b, i: (0, 0)),
        ],
        out_specs=pl.BlockSpec((1, TI, V), lambda b, i: (b, i, 0)),
        out_shape=jax.ShapeDtypeStruct((B, V, V), jnp.float32),
    )(zT, z, oht, ohh, WaT, Wb, lin3_b.reshape(1, H2), W4sel,
      out_b.reshape(1, 1))


def kernel(adjacency, node_features, edge_attributes,
           ec1_W1, ec1_b1, ec1_W2, ec1_b2,
           ec2_W1, ec2_b1, ec2_W2, ec2_b2,
           lin3_W, lin3_b, out_W, out_b):
    B, V, C = node_features.shape
    TI = _TI
    ea = jnp.concatenate([edge_attributes, adjacency[..., None]], axis=-1)
    r = jnp.arange(TI * V)
    oh = (r[:, None] // V == jnp.arange(TI)[None, :]).astype(jnp.float32)
    y = _edge_conv(ea, oh, node_features, ec1_W1, ec1_b1, ec1_W2, ec1_b2)
    z = _edge_conv(ea, oh, y, ec2_W1, ec2_b1, ec2_W2, ec2_b2)
    return _edge_predict(z, lin3_W, lin3_b, out_W, out_b)


# speedup vs baseline: 1.7378x; 1.5645x over previous
import functools

import jax
import jax.numpy as jnp
from jax.experimental import pallas as pl

_TI = 32


def _ec_body(TI, xT_ref, xq_ref, adj_ref, eT_ref,
             WabT_ref, WbT_ref, WcT_ref, b1T_ref, W2T_ref, b2T_ref, yq_ref):
    V = xT_ref.shape[2]
    H = W2T_ref.shape[0]
    xT = xT_ref[0]
    BnT = WbT_ref[...] @ xT
    AT = WabT_ref[...] @ xq_ref[0, 0] + b1T_ref[...]
    eT = eT_ref[0]
    adj = adj_ref[0]
    cols = []
    for t in range(TI):
        ET = WcT_ref[...] @ eT[t]
        h1 = jnp.maximum(ET + AT[:, t:t + 1] + BnT, 0.0)
        h2 = jnp.maximum(W2T_ref[...] @ h1 + b2T_ref[...], 0.0)
        pen = jnp.where(adj[t:t + 1] > 0.0, 0.0, jnp.float32(-1e30))
        agg = jnp.max(h2 + pen, axis=1, keepdims=True)
        cols.append(jnp.where(agg <= -1e29, 0.0, agg))
    yq_ref[0, 0] = jnp.concatenate(cols, axis=1)


def _edge_conv(adj, xT, xq, eT, W1, b1, W2, b2):
    B, C, V = xT.shape
    C2 = eT.shape[2]
    H = W2.shape[1]
    WabT = (W1[:C] - W1[C:2 * C]).T
    WbT = W1[C:2 * C].T
    WcT = W1[2 * C:].T
    TI = _TI
    return pl.pallas_call(
        functools.partial(_ec_body, TI),
        grid=(B, V // TI),
        in_specs=[
            pl.BlockSpec((1, C, V), lambda b, i: (b, 0, 0)),
            pl.BlockSpec((1, 1, C, TI), lambda b, i: (b, i, 0, 0)),
            pl.BlockSpec((1, TI, V), lambda b, i: (b, i, 0)),
            pl.BlockSpec((1, TI, C2, V), lambda b, i: (b, i, 0, 0)),
            pl.BlockSpec((H, C), lambda b, i: (0, 0)),
            pl.BlockSpec((H, C), lambda b, i: (0, 0)),
            pl.BlockSpec((H, C2), lambda b, i: (0, 0)),
            pl.BlockSpec((H, 1), lambda b, i: (0, 0)),
            pl.BlockSpec((H, H), lambda b, i: (0, 0)),
            pl.BlockSpec((H, 1), lambda b, i: (0, 0)),
        ],
        out_specs=pl.BlockSpec((1, 1, H, TI), lambda b, i: (b, i, 0, 0)),
        out_shape=jax.ShapeDtypeStruct((B, V // TI, H, TI), jnp.float32),
    )(xT, xq, adj, eT, WabT, WbT, WcT, b1.reshape(H, 1), W2.T,
      b2.reshape(H, 1))


def _fin_body(TI, zT_ref, zq_ref, oht_ref, ohh_ref,
              WaT_ref, WbT_ref, b3T_ref, W4sel_ref, b4_ref, out_ref):
    V = zT_ref.shape[2]
    H2 = WaT_ref.shape[0]
    PT = WaT_ref[...] @ zT_ref[0]
    QT = WbT_ref[...] @ zq_ref[0, 0] + b3T_ref[...]
    R1 = ohh_ref[...] @ QT
    Qcol = (R1 * oht_ref[...]) @ jnp.ones((TI, 1), jnp.float32)
    PTt = jnp.broadcast_to(PT[None], (TI, H2, V)).reshape(TI * H2, V)
    Hb = jnp.maximum(PTt + Qcol, 0.0)
    ot = W4sel_ref[...] @ Hb + b4_ref[...]
    out_ref[0] = jax.nn.sigmoid(ot)


def _edge_predict(zT, zq, lin3_W, lin3_b, out_W, out_b):
    B, C, V = zT.shape
    H2 = lin3_W.shape[1]
    TI = _TI
    WaT = lin3_W[:C].T
    WbT = lin3_W[C:].T
    r = jnp.arange(TI * H2)
    oht = (r[:, None] // H2 == jnp.arange(TI)[None, :]).astype(jnp.float32)
    ohh = (r[:, None] % H2 == jnp.arange(H2)[None, :]).astype(jnp.float32)
    W4sel = jnp.kron(jnp.eye(TI, dtype=jnp.float32), out_W.reshape(1, H2))
    return pl.pallas_call(
        functools.partial(_fin_body, TI),
        grid=(B, V // TI),
        in_specs=[
            pl.BlockSpec((1, C, V), lambda b, i: (b, 0, 0)),
            pl.BlockSpec((1, 1, C, TI), lambda b, i: (b, i, 0, 0)),
            pl.BlockSpec((TI * H2, TI), lambda b, i: (0, 0)),
            pl.BlockSpec((TI * H2, H2), lambda b, i: (0, 0)),
            pl.BlockSpec((H2, C), lambda b, i: (0, 0)),
            pl.BlockSpec((H2, C), lambda b, i: (0, 0)),
            pl.BlockSpec((H2, 1), lambda b, i: (0, 0)),
            pl.BlockSpec((TI, TI * H2), lambda b, i: (0, 0)),
            pl.BlockSpec((1, 1), lambda b, i: (0, 0)),
        ],
        out_specs=pl.BlockSpec((1, TI, V), lambda b, i: (b, i, 0)),
        out_shape=jax.ShapeDtypeStruct((B, V, V), jnp.float32),
    )(zT, zq, oht, ohh, WaT, WbT, lin3_b.reshape(H2, 1), W4sel,
      out_b.reshape(1, 1))


def _to_full(q):
    B, G, C, TI = q.shape
    return q.transpose(0, 2, 1, 3).reshape(B, C, G * TI)


def kernel(adjacency, node_features, edge_attributes,
           ec1_W1, ec1_b1, ec1_W2, ec1_b2,
           ec2_W1, ec2_b1, ec2_W2, ec2_b2,
           lin3_W, lin3_b, out_W, out_b):
    B, V, C = node_features.shape
    TI = _TI
    xT = node_features.transpose(0, 2, 1)
    xq = node_features.reshape(B, V // TI, TI, C).transpose(0, 1, 3, 2)
    eT = edge_attributes.transpose(0, 1, 3, 2)
    yq = _edge_conv(adjacency, xT, xq, eT, ec1_W1, ec1_b1, ec1_W2, ec1_b2)
    zq = _edge_conv(adjacency, _to_full(yq), yq, eT,
                    ec2_W1, ec2_b1, ec2_W2, ec2_b2)
    return _edge_predict(_to_full(zq), zq, lin3_W, lin3_b, out_W, out_b)


# speedup vs baseline: 3.5346x; 2.0340x over previous
import functools

import jax
import jax.numpy as jnp
from jax.experimental import pallas as pl

_TI = 32


_G = 2


def _ec_body(TI, G, H, xT_ref, xq_ref, adj_ref, eT_ref,
             WabT_ref, WbT_ref, W1blk_ref, b1T_ref, W2blk_ref, b2G_ref,
             yq_ref):
    V = xT_ref.shape[2]
    TC2 = W1blk_ref.shape[1]
    xT = xT_ref[0]
    BnT = WbT_ref[...] @ xT
    AT = WabT_ref[...] @ xq_ref[0, 0] + b1T_ref[...]
    eTs = eT_ref[0].reshape(TC2, V)
    big = W1blk_ref[...] @ eTs
    Acol = jnp.concatenate([AT[:, t:t + 1] for t in range(TI)], axis=0)
    Bnb = jnp.broadcast_to(BnT[None], (TI, H, V)).reshape(TI * H, V)
    h1 = jnp.maximum(big + Acol + Bnb, 0.0)
    pen = jnp.where(adj_ref[0] > 0.0, 0.0, jnp.float32(-1e30))
    cols = []
    for g in range(TI // G):
        h2 = jnp.maximum(
            W2blk_ref[...] @ h1[g * G * H:(g + 1) * G * H, :] + b2G_ref[...],
            0.0)
        for u in range(G):
            t = g * G + u
            hm = h2[u * H:(u + 1) * H, :] + pen[t:t + 1, :]
            agg = jnp.max(hm, axis=1, keepdims=True)
            cols.append(jnp.where(agg <= -1e29, 0.0, agg))
    yq_ref[0, 0] = jnp.concatenate(cols, axis=1)


def _edge_conv(adj, xT, xq, eT, W1, b1, W2, b2):
    B, C, V = xT.shape
    C2 = eT.shape[2]
    H = W2.shape[1]
    WabT = (W1[:C] - W1[C:2 * C]).T
    WbT = W1[C:2 * C].T
    WcT = W1[2 * C:].T
    TI, G = _TI, _G
    eye = jnp.eye(TI, dtype=jnp.float32)
    W1blk = jnp.kron(eye, WcT)
    W2blk = jnp.kron(jnp.eye(G, dtype=jnp.float32), W2.T)
    b2G = jnp.tile(b2.reshape(H, 1), (G, 1))
    return pl.pallas_call(
        functools.partial(_ec_body, TI, G, H),
        grid=(B, V // TI),
        in_specs=[
            pl.BlockSpec((1, C, V), lambda b, i: (b, 0, 0)),
            pl.BlockSpec((1, 1, C, TI), lambda b, i: (b, i, 0, 0)),
            pl.BlockSpec((1, TI, V), lambda b, i: (b, i, 0)),
            pl.BlockSpec((1, TI, C2, V), lambda b, i: (b, i, 0, 0)),
            pl.BlockSpec((H, C), lambda b, i: (0, 0)),
            pl.BlockSpec((H, C), lambda b, i: (0, 0)),
            pl.BlockSpec((TI * H, TI * C2), lambda b, i: (0, 0)),
            pl.BlockSpec((H, 1), lambda b, i: (0, 0)),
            pl.BlockSpec((G * H, G * H), lambda b, i: (0, 0)),
            pl.BlockSpec((G * H, 1), lambda b, i: (0, 0)),
        ],
        out_specs=pl.BlockSpec((1, 1, H, TI), lambda b, i: (b, i, 0, 0)),
        out_shape=jax.ShapeDtypeStruct((B, V // TI, H, TI), jnp.float32),
    )(xT, xq, adj, eT, WabT, WbT, W1blk, b1.reshape(H, 1), W2blk, b2G)


def _fin_body(TI, zT_ref, zq_ref, oht_ref, ohh_ref,
              WaT_ref, WbT_ref, b3T_ref, W4sel_ref, b4_ref, out_ref):
    V = zT_ref.shape[2]
    H2 = WaT_ref.shape[0]
    PT = WaT_ref[...] @ zT_ref[0]
    QT = WbT_ref[...] @ zq_ref[0, 0] + b3T_ref[...]
    R1 = ohh_ref[...] @ QT
    Qcol = (R1 * oht_ref[...]) @ jnp.ones((TI, 1), jnp.float32)
    PTt = jnp.broadcast_to(PT[None], (TI, H2, V)).reshape(TI * H2, V)
    Hb = jnp.maximum(PTt + Qcol, 0.0)
    ot = W4sel_ref[...] @ Hb + b4_ref[...]
    out_ref[0] = jax.nn.sigmoid(ot)


def _edge_predict(zT, zq, lin3_W, lin3_b, out_W, out_b):
    B, C, V = zT.shape
    H2 = lin3_W.shape[1]
    TI = _TI
    WaT = lin3_W[:C].T
    WbT = lin3_W[C:].T
    r = jnp.arange(TI * H2)
    oht = (r[:, None] // H2 == jnp.arange(TI)[None, :]).astype(jnp.float32)
    ohh = (r[:, None] % H2 == jnp.arange(H2)[None, :]).astype(jnp.float32)
    W4sel = jnp.kron(jnp.eye(TI, dtype=jnp.float32), out_W.reshape(1, H2))
    return pl.pallas_call(
        functools.partial(_fin_body, TI),
        grid=(B, V // TI),
        in_specs=[
            pl.BlockSpec((1, C, V), lambda b, i: (b, 0, 0)),
            pl.BlockSpec((1, 1, C, TI), lambda b, i: (b, i, 0, 0)),
            pl.BlockSpec((TI * H2, TI), lambda b, i: (0, 0)),
            pl.BlockSpec((TI * H2, H2), lambda b, i: (0, 0)),
            pl.BlockSpec((H2, C), lambda b, i: (0, 0)),
            pl.BlockSpec((H2, C), lambda b, i: (0, 0)),
            pl.BlockSpec((H2, 1), lambda b, i: (0, 0)),
            pl.BlockSpec((TI, TI * H2), lambda b, i: (0, 0)),
            pl.BlockSpec((1, 1), lambda b, i: (0, 0)),
        ],
        out_specs=pl.BlockSpec((1, TI, V), lambda b, i: (b, i, 0)),
        out_shape=jax.ShapeDtypeStruct((B, V, V), jnp.float32),
    )(zT, zq, oht, ohh, WaT, WbT, lin3_b.reshape(H2, 1), W4sel,
      out_b.reshape(1, 1))


def _to_full(q):
    B, G, C, TI = q.shape
    return q.transpose(0, 2, 1, 3).reshape(B, C, G * TI)


def kernel(adjacency, node_features, edge_attributes,
           ec1_W1, ec1_b1, ec1_W2, ec1_b2,
           ec2_W1, ec2_b1, ec2_W2, ec2_b2,
           lin3_W, lin3_b, out_W, out_b):
    B, V, C = node_features.shape
    TI = _TI
    xT = node_features.transpose(0, 2, 1)
    xq = node_features.reshape(B, V // TI, TI, C).transpose(0, 1, 3, 2)
    eT = edge_attributes.transpose(0, 1, 3, 2)
    yq = _edge_conv(adjacency, xT, xq, eT, ec1_W1, ec1_b1, ec1_W2, ec1_b2)
    zq = _edge_conv(adjacency, _to_full(yq), yq, eT,
                    ec2_W1, ec2_b1, ec2_W2, ec2_b2)
    return _edge_predict(_to_full(zq), zq, lin3_W, lin3_b, out_W, out_b)
